# Initial kernel scaffold; baseline (speedup 1.0000x reference)
import jax
import jax.numpy as jnp
from jax.experimental import pallas as pl


def kernel(data):
    raise NotImplementedError("write your pallas kernel here")


# speedup vs baseline: 2.0365x; 2.0365x over previous
import functools

import jax
import jax.numpy as jnp
import numpy as np
from jax import lax
from jax.experimental import pallas as pl
from jax.experimental.pallas import tpu as pltpu
from jax.experimental.pallas import tpu_sc as plsc

_N = 100000
_D = 128
_NC = 2
_NS = 16
_NW = _NC * _NS
_C = 128
_NCHUNK = 25
_B_PER_W = _C * _NCHUNK
_B_PAD = _B_PER_W * _NW

_PERM_CACHE: dict = {}


def _perm_padded() -> np.ndarray:
    if "p" not in _PERM_CACHE:
        with jax.ensure_compile_time_eval():
            p = jax.random.permutation(jax.random.key(42), _N)
        p = np.asarray(p, dtype=np.int32)
        _PERM_CACHE["p"] = np.concatenate(
            [p, np.zeros((_B_PAD - _N,), dtype=np.int32)])
    return _PERM_CACHE["p"]


@functools.partial(
    pl.kernel,
    out_type=jax.ShapeDtypeStruct((_B_PAD, _D), jnp.float32),
    mesh=plsc.VectorSubcoreMesh(core_axis_name="c", subcore_axis_name="s"),
    scratch_types=[
        pltpu.VMEM((_NCHUNK, _C), jnp.int32),
        pltpu.VMEM((2, _C, _D), jnp.float32),
        pltpu.SemaphoreType.DMA,
        pltpu.SemaphoreType.DMA,
    ],
)
def _permute_rows(table_hbm, idx_hbm, out_hbm, idx_v, bufs, gsem, ssem):
    wid = lax.axis_index("s") * _NC + lax.axis_index("c")
    base = wid * _B_PER_W
    pltpu.sync_copy(idx_hbm.at[wid], idx_v)

    def gather(j, b):
        return pltpu.async_copy(table_hbm.at[idx_v.at[j]], bufs.at[b], gsem)

    def store(j, b):
        return pltpu.async_copy(
            bufs.at[b], out_hbm.at[pl.ds(base + j * _C, _C)], ssem)

    gathers = [None, None]
    stores = [None, None]
    gathers[0] = gather(0, 0)
    for j in range(_NCHUNK):
        b = j % 2
        nb = (j + 1) % 2
        if j + 1 < _NCHUNK:
            if stores[nb] is not None:
                stores[nb].wait()
            gathers[nb] = gather(j + 1, nb)
        gathers[b].wait()
        stores[b] = store(j, b)
    for s in stores:
        if s is not None:
            s.wait()


def kernel(data):
    x = data.reshape(_N, _D)
    idx = jnp.asarray(_perm_padded()).reshape(_NW, _NCHUNK, _C)
    out = _permute_rows(x, idx)
    return out[:_N].reshape(_N * _D)


# speedup vs baseline: 2.0889x; 1.0257x over previous
import functools

import jax
import jax.numpy as jnp
import numpy as np
from jax import lax
from jax.experimental import pallas as pl
from jax.experimental.pallas import tpu as pltpu
from jax.experimental.pallas import tpu_sc as plsc

_N = 100000
_D = 128
_NC = 2
_NS = 16
_NW = _NC * _NS
_C = 128
_NCHUNK = 25
_B_PER_W = _C * _NCHUNK
_B_PAD = _B_PER_W * _NW

_PERM_CACHE: dict = {}


def _perm_padded() -> np.ndarray:
    if "p" not in _PERM_CACHE:
        with jax.ensure_compile_time_eval():
            p = jax.random.permutation(jax.random.key(42), _N)
        p = np.asarray(p, dtype=np.int32)
        _PERM_CACHE["p"] = np.concatenate(
            [p, np.zeros((_B_PAD - _N,), dtype=np.int32)])
    return _PERM_CACHE["p"]


@functools.partial(
    pl.kernel,
    out_type=jax.ShapeDtypeStruct((_B_PAD, _D), jnp.float32),
    mesh=plsc.VectorSubcoreMesh(core_axis_name="c", subcore_axis_name="s"),
    scratch_types=[
        pltpu.VMEM((_NCHUNK, _C), jnp.int32),
        pltpu.VMEM((6, _C, _D), jnp.float32),
        pltpu.SemaphoreType.DMA,
        pltpu.SemaphoreType.DMA,
    ],
)
def _permute_rows(table_hbm, idx_hbm, out_hbm, idx_v, bufs, gsem, ssem):
    wid = lax.axis_index("s") * _NC + lax.axis_index("c")
    base = wid * _B_PER_W
    pltpu.sync_copy(idx_hbm.at[wid], idx_v)

    _NBUF = 6
    _G = 3

    def gather(j):
        return pltpu.async_copy(
            table_hbm.at[idx_v.at[j]], bufs.at[j % _NBUF], gsem)

    def store(j):
        return pltpu.async_copy(
            bufs.at[j % _NBUF], out_hbm.at[pl.ds(base + j * _C, _C)], ssem)

    gathers = {}
    stores = {}
    for j in range(min(_G, _NCHUNK)):
        gathers[j] = gather(j)
    for j in range(_NCHUNK):
        nxt = j + _G
        if nxt < _NCHUNK:
            prev = nxt - _NBUF
            if prev >= 0:
                stores.pop(prev).wait()
            gathers[nxt] = gather(nxt)
        gathers.pop(j).wait()
        stores[j] = store(j)
    for j in sorted(stores):
        stores.pop(j).wait()


def kernel(data):
    x = data.reshape(_N, _D)
    idx = jnp.asarray(_perm_padded()).reshape(_NW, _NCHUNK, _C)
    out = _permute_rows(x, idx)
    return out[:_N].reshape(_N * _D)


# speedup vs baseline: 6.7391x; 3.2262x over previous
import functools

import jax
import jax.numpy as jnp
import numpy as np
from jax import lax
from jax.experimental import pallas as pl
from jax.experimental.pallas import tpu as pltpu
from jax.experimental.pallas import tpu_sc as plsc

_N = 100000
_D = 128
_NC = 2
_NS = 16
_NW = _NC * _NS
_C = 128
_NFULL = _N // _C
_TAIL = _N - _NFULL * _C
_KMAX = 25
_NLONG = _NFULL - 24 * _NW

_PERM_CACHE: dict = {}


def _chunk_start(w: int):
    return _KMAX * w if w < _NLONG else 24 * w + _NLONG


def _perm_chunked() -> np.ndarray:
    if "p" not in _PERM_CACHE:
        with jax.ensure_compile_time_eval():
            p = jax.random.permutation(jax.random.key(42), _N)
        p = np.asarray(p, dtype=np.int32)
        idx3 = np.zeros((_NW, _KMAX, _C), dtype=np.int32)
        for w in range(_NW):
            s = _chunk_start(w)
            k_w = _KMAX if w < _NLONG else 24
            idx3[w, :k_w, :] = p[_C * s: _C * (s + k_w)].reshape(k_w, _C)
        idx3[_NW - 1, 24, :_TAIL] = p[_NFULL * _C:]
        _PERM_CACHE["p"] = idx3
    return _PERM_CACHE["p"]


@functools.partial(
    pl.kernel,
    out_type=jax.ShapeDtypeStruct((_N, _D), jnp.float32),
    mesh=plsc.VectorSubcoreMesh(core_axis_name="c", subcore_axis_name="s"),
    scratch_types=[
        pltpu.VMEM((_KMAX, _C), jnp.int32),
        pltpu.VMEM((6, _C, _D), jnp.float32),
        pltpu.SemaphoreType.DMA,
        pltpu.SemaphoreType.DMA,
    ],
)
def _permute_rows(table_hbm, idx_hbm, out_hbm, idx_v, bufs, gsem, ssem):
    wid = lax.axis_index("s") * _NC + lax.axis_index("c")
    start = jnp.where(wid < _NLONG, _KMAX * wid, 24 * wid + _NLONG)
    pltpu.sync_copy(idx_hbm.at[wid], idx_v)

    _NBUF = 6
    _G = 3

    def gather(k):
        return pltpu.async_copy(
            table_hbm.at[idx_v.at[k]], bufs.at[k % _NBUF], gsem)

    def store(k):
        return pltpu.async_copy(
            bufs.at[k % _NBUF],
            out_hbm.at[pl.ds((start + k) * _C, _C)], ssem)

    gathers = {}
    stores = {}
    for k in range(_G):
        gathers[k] = gather(k)
    for k in range(24):
        nxt = k + _G
        if nxt < 24:
            prev = nxt - _NBUF
            if prev >= 0:
                stores.pop(prev).wait()
            gathers[nxt] = gather(nxt)
        gathers.pop(k).wait()
        stores[k] = store(k)
    for k in sorted(stores):
        stores.pop(k).wait()

    @pl.when(wid < _NLONG)
    def _():
        gather(24).wait()
        store(24).wait()

    @pl.when(wid == _NW - 1)
    def _():
        pltpu.async_copy(
            table_hbm.at[idx_v.at[24, pl.ds(0, _TAIL)]],
            bufs.at[0, pl.ds(0, _TAIL)], gsem).wait()
        pltpu.async_copy(
            bufs.at[0, pl.ds(0, _TAIL)],
            out_hbm.at[pl.ds(_NFULL * _C, _TAIL)], ssem).wait()


def kernel(data):
    x = data.reshape(_N, _D)
    idx = jnp.asarray(_perm_chunked())
    out = _permute_rows(x, idx)
    return out.reshape(_N * _D)
